# Initial kernel scaffold; baseline (speedup 1.0000x reference)
import jax
import jax.numpy as jnp
from jax.experimental import pallas as pl


def kernel(weights1, weights2, bias1, bias2, vals, hor_rows, hor_cols, ver_rows, ver_cols):
    raise NotImplementedError("write your pallas kernel here")


# speedup vs baseline: 7.1378x; 7.1378x over previous
import functools

import jax
import jax.numpy as jnp
from jax import lax
from jax.experimental import pallas as pl
from jax.experimental.pallas import tpu as pltpu
from jax.experimental.pallas import tpu_sc as plsc

_EMB = 16
_NC, _NS = 2, 16
_NW = _NC * _NS
_C = 128


def _sc_spmm(table, cols, rows, vals, zeros, n):
    kp = cols.shape[0]
    nchunk = kp // (_NW * _C)
    slab = n // _NS

    mesh = plsc.VectorSubcoreMesh(core_axis_name="c", subcore_axis_name="s")

    @functools.partial(
        pl.kernel,
        out_type=jax.ShapeDtypeStruct((_NC, _NS, slab, _EMB), jnp.float32),
        mesh=mesh,
        scratch_types=[
            pltpu.VMEM_SHARED((n, _EMB), jnp.float32),
            pltpu.VMEM((_C,), jnp.int32),
            pltpu.VMEM((_C,), jnp.int32),
            pltpu.VMEM((_C,), jnp.float32),
            pltpu.VMEM((_C, _EMB), jnp.float32),
            pltpu.SemaphoreType.DMA,
        ],
        compiler_params=pltpu.CompilerParams(use_tc_tiling_on_sc=False),
    )
    def spmm(table_r, cols_r, rows_r, vals_r, zeros_r, out_r,
             acc, idx_v, row_v, val_v, msg_v, sem):
        cid = lax.axis_index("c")
        sid = lax.axis_index("s")
        wid = sid * _NC + cid
        pltpu.sync_copy(zeros_r.at[sid],
                        acc.at[pl.ds(sid * slab, slab), :])
        plsc.subcore_barrier()

        def chunk(j, carry):
            base = (wid * nchunk + j) * _C
            pltpu.sync_copy(cols_r.at[pl.ds(base, _C)], idx_v)
            pltpu.sync_copy(rows_r.at[pl.ds(base, _C)], row_v)
            pltpu.sync_copy(vals_r.at[pl.ds(base, _C)], val_v)
            pltpu.async_copy(table_r.at[idx_v], msg_v, sem).wait()

            def mul(b, c2):
                vv = val_v[pl.ds(b * 16, 16)]
                for l in range(16):
                    i = b * 16 + l
                    msg_v[i, :] = msg_v[i, :] * vv[l]
                return c2
            lax.fori_loop(0, _C // 16, mul, 0)
            pltpu.sync_copy(msg_v, acc.at[row_v], add=True)
            return carry

        lax.fori_loop(0, nchunk, chunk, 0)
        plsc.subcore_barrier()
        pltpu.sync_copy(acc.at[pl.ds(sid * slab, slab), :],
                        out_r.at[cid, sid])

    out = spmm(table, cols, rows, vals, zeros.reshape(_NS, slab, _EMB))
    return out.reshape(_NC, n, _EMB)


def _mid_tc(hp, b1, w2, n):
    bn = 2000
    nb = n // bn
    r = w2.shape[0]

    def body(hp_ref, b1_ref, w2_ref, g_ref):
        h = jnp.maximum(hp_ref[0] + hp_ref[1] + b1_ref[:, :], 0.0)
        g_ref[:, :] = jnp.dot(h, w2_ref[0], preferred_element_type=jnp.float32)

    return pl.pallas_call(
        body,
        grid=(nb, r),
        in_specs=[
            pl.BlockSpec((2, bn, _EMB), lambda i, j: (0, i, 0)),
            pl.BlockSpec((1, _EMB), lambda i, j: (0, 0)),
            pl.BlockSpec((1, _EMB, _EMB), lambda i, j: (j, 0, 0)),
        ],
        out_specs=pl.BlockSpec((bn, _EMB), lambda i, j: (j * nb + i, 0)),
        out_shape=jax.ShapeDtypeStruct((r * n, _EMB), jnp.float32),
    )(hp, b1.reshape(1, _EMB), w2)


def _final_tc(op, b2, n):
    bn = 5000

    def body(p_ref, b2_ref, o_ref):
        o_ref[:, :] = p_ref[0] + p_ref[1] + b2_ref[:, :]

    return pl.pallas_call(
        body,
        grid=(n // bn,),
        in_specs=[
            pl.BlockSpec((2, bn, _EMB), lambda i: (0, i, 0)),
            pl.BlockSpec((1, _EMB), lambda i: (0, 0)),
        ],
        out_specs=pl.BlockSpec((bn, _EMB), lambda i: (i, 0)),
        out_shape=jax.ShapeDtypeStruct((n, _EMB), jnp.float32),
    )(op, b2.reshape(1, _EMB))


def kernel(weights1, weights2, bias1, bias2, vals,
           hor_rows, hor_cols, ver_rows, ver_cols):
    r, n, emb = weights1.shape
    k = hor_rows.shape[0]
    wflat = weights1.reshape(r * n, emb)

    cpb = _NW * _C
    kp = ((k + cpb - 1) // cpb) * cpb
    pad = kp - k
    cols = jnp.concatenate([hor_cols.astype(jnp.int32),
                            jnp.zeros((pad,), jnp.int32)])
    rows = jnp.concatenate([hor_rows.astype(jnp.int32),
                            jnp.zeros((pad,), jnp.int32)])
    v = jnp.concatenate([vals.astype(jnp.float32),
                         jnp.zeros((pad,), jnp.float32)])
    zeros = jnp.zeros((n, emb), jnp.float32)

    hp = _sc_spmm(wflat, cols, rows, v, zeros, n)
    g = _mid_tc(hp, bias1, weights2, n)
    op = _sc_spmm(g, cols, rows, v, zeros, n)
    return _final_tc(op, bias2, n)


# speedup vs baseline: 13.8224x; 1.9365x over previous
import functools

import jax
import jax.numpy as jnp
from jax import lax
from jax.experimental import pallas as pl
from jax.experimental.pallas import tpu as pltpu
from jax.experimental.pallas import tpu_sc as plsc

_EMB = 16
_NC, _NS = 2, 16
_NW = _NC * _NS
_C = 128


_NB = 8
_MC = _NB * _C


def _sc_spmm(table, cols, rows, vals, zeros, n):
    m_total = cols.shape[0]
    nm = m_total // _NW
    slab = n // _NS

    mesh = plsc.VectorSubcoreMesh(core_axis_name="c", subcore_axis_name="s")

    @functools.partial(
        pl.kernel,
        out_type=jax.ShapeDtypeStruct((_NC, _NS, slab, _EMB), jnp.float32),
        mesh=mesh,
        scratch_types=[
            pltpu.VMEM_SHARED((n, _EMB), jnp.float32),
            pltpu.VMEM((2, _NB, _C), jnp.int32),
            pltpu.VMEM((2, _NB, _C), jnp.int32),
            pltpu.VMEM((2, _NB, _C), jnp.float32),
            pltpu.VMEM((_MC, _EMB), jnp.float32),
            pltpu.SemaphoreType.DMA,
            pltpu.SemaphoreType.DMA,
            pltpu.SemaphoreType.DMA,
        ],
        compiler_params=pltpu.CompilerParams(use_tc_tiling_on_sc=False),
    )
    def spmm(table_r, cols_r, rows_r, vals_r, zeros_r, out_r,
             acc, idx_v, row_v, val_v, msg_v, sem_in, sem_g, sem_s):
        cid = lax.axis_index("c")
        sid = lax.axis_index("s")
        wid = sid * _NC + cid
        m0 = wid * nm
        pltpu.sync_copy(zeros_r.at[sid],
                        acc.at[pl.ds(sid * slab, slab), :])
        plsc.subcore_barrier()

        def fire_inputs(m, p):
            pltpu.async_copy(cols_r.at[m], idx_v.at[p], sem_in)
            pltpu.async_copy(rows_r.at[m], row_v.at[p], sem_in)
            pltpu.async_copy(vals_r.at[m], val_v.at[p], sem_in)

        def wait_inputs(p):
            pltpu.make_async_copy(cols_r.at[0], idx_v.at[p], sem_in).wait()
            pltpu.make_async_copy(rows_r.at[0], row_v.at[p], sem_in).wait()
            pltpu.make_async_copy(vals_r.at[0], val_v.at[p], sem_in).wait()

        fire_inputs(m0, 0)

        def chunk(j, carry):
            p = lax.rem(j, 2)
            wait_inputs(p)
            mnext = jnp.where(j + 1 < nm, m0 + j + 1, m0 + j)
            fire_inputs(mnext, 1 - p)
            gd = [pltpu.async_copy(table_r.at[idx_v.at[p, b]],
                                   msg_v.at[pl.ds(b * _C, _C), :], sem_g)
                  for b in range(_NB)]
            sd = []
            for b in range(_NB):
                gd[b].wait()

                def mul(q, c2, b=b):
                    vv = val_v[p, b, pl.ds(q * 16, 16)]
                    for l in range(16):
                        i = b * _C + q * 16 + l
                        msg_v[i, :] = msg_v[i, :] * vv[l]
                    return c2
                lax.fori_loop(0, _C // 16, mul, 0)
                sd.append(pltpu.async_copy(msg_v.at[pl.ds(b * _C, _C), :],
                                           acc.at[row_v.at[p, b]],
                                           sem_s, add=True))
            for b in range(_NB):
                sd[b].wait()
            return carry

        lax.fori_loop(0, nm, chunk, 0)
        wait_inputs(lax.rem(jnp.int32(nm), 2))
        plsc.subcore_barrier()
        pltpu.sync_copy(acc.at[pl.ds(sid * slab, slab), :],
                        out_r.at[cid, sid])

    out = spmm(table, cols, rows, vals, zeros.reshape(_NS, slab, _EMB))
    return out.reshape(_NC, n, _EMB)


def _mid_tc(hp, b1, w2, n):
    bn = 2000
    nb = n // bn
    r = w2.shape[0]

    def body(hp_ref, b1_ref, w2_ref, g_ref):
        h = jnp.maximum(hp_ref[0] + hp_ref[1] + b1_ref[:, :], 0.0)
        g_ref[:, :] = jnp.dot(h, w2_ref[0], preferred_element_type=jnp.float32)

    return pl.pallas_call(
        body,
        grid=(nb, r),
        in_specs=[
            pl.BlockSpec((2, bn, _EMB), lambda i, j: (0, i, 0)),
            pl.BlockSpec((1, _EMB), lambda i, j: (0, 0)),
            pl.BlockSpec((1, _EMB, _EMB), lambda i, j: (j, 0, 0)),
        ],
        out_specs=pl.BlockSpec((bn, _EMB), lambda i, j: (j * nb + i, 0)),
        out_shape=jax.ShapeDtypeStruct((r * n, _EMB), jnp.float32),
    )(hp, b1.reshape(1, _EMB), w2)


def _final_tc(op, b2, n):
    bn = 5000

    def body(p_ref, b2_ref, o_ref):
        o_ref[:, :] = p_ref[0] + p_ref[1] + b2_ref[:, :]

    return pl.pallas_call(
        body,
        grid=(n // bn,),
        in_specs=[
            pl.BlockSpec((2, bn, _EMB), lambda i: (0, i, 0)),
            pl.BlockSpec((1, _EMB), lambda i: (0, 0)),
        ],
        out_specs=pl.BlockSpec((bn, _EMB), lambda i: (i, 0)),
        out_shape=jax.ShapeDtypeStruct((n, _EMB), jnp.float32),
    )(op, b2.reshape(1, _EMB))


def kernel(weights1, weights2, bias1, bias2, vals,
           hor_rows, hor_cols, ver_rows, ver_cols):
    r, n, emb = weights1.shape
    k = hor_rows.shape[0]
    wflat = weights1.reshape(r * n, emb)

    cpb = _NW * _MC
    kp = ((k + cpb - 1) // cpb) * cpb
    pad = kp - k
    cols = jnp.concatenate([hor_cols.astype(jnp.int32),
                            jnp.zeros((pad,), jnp.int32)]).reshape(-1, _NB, _C)
    rows = jnp.concatenate([hor_rows.astype(jnp.int32),
                            jnp.zeros((pad,), jnp.int32)]).reshape(-1, _NB, _C)
    v = jnp.concatenate([vals.astype(jnp.float32),
                         jnp.zeros((pad,), jnp.float32)]).reshape(-1, _NB, _C)
    zeros = jnp.zeros((n, emb), jnp.float32)

    hp = _sc_spmm(wflat, cols, rows, v, zeros, n)
    g = _mid_tc(hp, bias1, weights2, n)
    op = _sc_spmm(g, cols, rows, v, zeros, n)
    return _final_tc(op, bias2, n)


# speedup vs baseline: 20.6758x; 1.4958x over previous
import functools

import jax
import jax.numpy as jnp
from jax import lax
from jax.experimental import pallas as pl
from jax.experimental.pallas import tpu as pltpu
from jax.experimental.pallas import tpu_sc as plsc

_EMB = 16
_NC, _NS = 2, 16
_NW = _NC * _NS
_C = 128


_NB = 8
_MC = _NB * _C


def _sc_spmm(table, cols, rows, vals, zeros, n):
    m_total = cols.shape[0]
    nm = m_total // _NW
    slab = n // _NS

    mesh = plsc.VectorSubcoreMesh(core_axis_name="c", subcore_axis_name="s")

    @functools.partial(
        pl.kernel,
        out_type=jax.ShapeDtypeStruct((_NC, _NS, slab, _EMB), jnp.float32),
        mesh=mesh,
        scratch_types=[
            pltpu.VMEM_SHARED((n, _EMB), jnp.float32),
            pltpu.VMEM((2, _NB, _C), jnp.int32),
            pltpu.VMEM((2, _NB, _C), jnp.int32),
            pltpu.VMEM((2, _NB, _C), jnp.float32),
            pltpu.VMEM((_MC, _EMB), jnp.float32),
            pltpu.SemaphoreType.DMA,
            pltpu.SemaphoreType.DMA,
            pltpu.SemaphoreType.DMA,
        ],
        compiler_params=pltpu.CompilerParams(use_tc_tiling_on_sc=False),
    )
    def spmm(table_r, cols_r, rows_r, vals_r, zeros_r, out_r,
             acc, idx_v, row_v, val_v, msg_v, sem_in, sem_g, sem_s):
        cid = lax.axis_index("c")
        sid = lax.axis_index("s")
        wid = sid * _NC + cid
        m0 = wid * nm
        pltpu.sync_copy(zeros_r.at[sid],
                        acc.at[pl.ds(sid * slab, slab), :])
        plsc.subcore_barrier()

        def fire_inputs(m, p):
            pltpu.async_copy(cols_r.at[m], idx_v.at[p], sem_in)
            pltpu.async_copy(rows_r.at[m], row_v.at[p], sem_in)
            pltpu.async_copy(vals_r.at[m], val_v.at[p], sem_in)

        def wait_inputs(p):
            pltpu.make_async_copy(cols_r.at[0], idx_v.at[p], sem_in).wait()
            pltpu.make_async_copy(rows_r.at[0], row_v.at[p], sem_in).wait()
            pltpu.make_async_copy(vals_r.at[0], val_v.at[p], sem_in).wait()

        fire_inputs(m0, 0)

        def chunk(j, carry):
            p = lax.rem(j, 2)
            wait_inputs(p)
            mnext = jnp.where(j + 1 < nm, m0 + j + 1, m0 + j)
            fire_inputs(mnext, 1 - p)
            gd = [pltpu.async_copy(table_r.at[idx_v.at[p, b]],
                                   msg_v.at[pl.ds(b * _C, _C), :], sem_g)
                  for b in range(_NB)]
            sd = []
            for b in range(_NB):
                gd[b].wait()

                def mul(q, c2, b=b):
                    vv = val_v[p, b, pl.ds(q * 16, 16)]
                    for l in range(16):
                        i = b * _C + q * 16 + l
                        msg_v[i, :] = msg_v[i, :] * vv[l]
                    return c2
                lax.fori_loop(0, _C // 16, mul, 0)
                sd.append(pltpu.async_copy(msg_v.at[pl.ds(b * _C, _C), :],
                                           acc.at[row_v.at[p, b]],
                                           sem_s, add=True))
            for b in range(_NB):
                sd[b].wait()
            return carry

        lax.fori_loop(0, nm, chunk, 0)
        wait_inputs(lax.rem(jnp.int32(nm), 2))
        plsc.subcore_barrier()
        pltpu.sync_copy(acc.at[pl.ds(sid * slab, slab), :],
                        out_r.at[cid, sid])

    out = spmm(table, cols, rows, vals, zeros.reshape(_NS, slab, _EMB))
    return out.reshape(_NC, n, _EMB)


def _mid_tc(hp8, b1t, w2bd, n):
    n8 = n // 8
    r = w2bd.shape[0]

    def body(hp_ref, b1_ref, w2_ref, g_ref):
        h = jnp.maximum(hp_ref[0] + hp_ref[1] + b1_ref[:, :], 0.0)
        g_ref[0] = jnp.dot(h, w2_ref[0], preferred_element_type=jnp.float32)

    return pl.pallas_call(
        body,
        grid=(r,),
        in_specs=[
            pl.BlockSpec((2, n8, 128), lambda j: (0, 0, 0)),
            pl.BlockSpec((1, 128), lambda j: (0, 0)),
            pl.BlockSpec((1, 128, 128), lambda j: (j, 0, 0)),
        ],
        out_specs=pl.BlockSpec((1, n8, 128), lambda j: (j, 0, 0)),
        out_shape=jax.ShapeDtypeStruct((r, n8, 128), jnp.float32),
    )(hp8, b1t.reshape(1, 128), w2bd)


def _final_tc(op8, b2t, n):
    n8 = n // 8

    def body(p_ref, b2_ref, o_ref):
        o_ref[:, :] = p_ref[0] + p_ref[1] + b2_ref[:, :]

    return pl.pallas_call(
        body,
        grid=(1,),
        in_specs=[
            pl.BlockSpec((2, n8, 128), lambda i: (0, 0, 0)),
            pl.BlockSpec((1, 128), lambda i: (0, 0)),
        ],
        out_specs=pl.BlockSpec((n8, 128), lambda i: (0, 0)),
        out_shape=jax.ShapeDtypeStruct((n8, 128), jnp.float32),
    )(op8, b2t.reshape(1, 128))


def kernel(weights1, weights2, bias1, bias2, vals,
           hor_rows, hor_cols, ver_rows, ver_cols):
    r, n, emb = weights1.shape
    k = hor_rows.shape[0]
    wflat = weights1.reshape(r * n, emb)

    cpb = _NW * _MC
    kp = ((k + cpb - 1) // cpb) * cpb
    pad = kp - k
    cols = jnp.concatenate([hor_cols.astype(jnp.int32),
                            jnp.zeros((pad,), jnp.int32)]).reshape(-1, _NB, _C)
    rows = jnp.concatenate([hor_rows.astype(jnp.int32),
                            jnp.zeros((pad,), jnp.int32)]).reshape(-1, _NB, _C)
    v = jnp.concatenate([vals.astype(jnp.float32),
                         jnp.zeros((pad,), jnp.float32)]).reshape(-1, _NB, _C)
    zeros = jnp.zeros((n, emb), jnp.float32)
    b1t = jnp.tile(bias1, 8)
    b2t = jnp.tile(bias2, 8)
    w2bd = jax.vmap(lambda w: jnp.kron(jnp.eye(8, dtype=w.dtype), w))(weights2)

    hp = _sc_spmm(wflat, cols, rows, v, zeros, n)
    g8 = _mid_tc(hp.reshape(_NC, n // 8, 128), b1t, w2bd, n)
    op = _sc_spmm(g8.reshape(r * n, emb), cols, rows, v, zeros, n)
    out8 = _final_tc(op.reshape(_NC, n // 8, 128), b2t, n)
    return out8.reshape(n, emb)


# speedup vs baseline: 27.3565x; 1.3231x over previous
import functools

import jax
import jax.numpy as jnp
from jax import lax
from jax.experimental import pallas as pl
from jax.experimental.pallas import tpu as pltpu
from jax.experimental.pallas import tpu_sc as plsc

_EMB = 16
_NC, _NS = 2, 16
_NW = _NC * _NS
_C = 128


_NB = 8
_MC = _NB * _C


def _sc_spmm(table, cols, rows, vals, zeros, n):
    m_total = cols.shape[0]
    nm = m_total // _NW
    slab = n // _NS

    mesh = plsc.VectorSubcoreMesh(core_axis_name="c", subcore_axis_name="s")

    @functools.partial(
        pl.kernel,
        out_type=jax.ShapeDtypeStruct((_NC, _NS, slab, _EMB), jnp.float32),
        mesh=mesh,
        scratch_types=[
            pltpu.VMEM_SHARED((n, _EMB), jnp.float32),
            pltpu.VMEM((2, _NB, _C), jnp.int32),
            pltpu.VMEM((2, _NB, _C), jnp.int32),
            pltpu.VMEM((2, _NB, _C), jnp.float32),
            pltpu.VMEM((_MC, _EMB), jnp.float32),
            pltpu.SemaphoreType.DMA,
            pltpu.SemaphoreType.DMA,
            pltpu.SemaphoreType.DMA,
        ],
        compiler_params=pltpu.CompilerParams(use_tc_tiling_on_sc=False),
    )
    def spmm(table_r, cols_r, rows_r, vals_r, zeros_r, out_r,
             acc, idx_v, row_v, val_v, msg_v, sem_in, sem_g, sem_s):
        cid = lax.axis_index("c")
        sid = lax.axis_index("s")
        wid = sid * _NC + cid
        m0 = wid * nm
        pltpu.sync_copy(zeros_r.at[sid],
                        acc.at[pl.ds(sid * slab, slab), :])
        plsc.subcore_barrier()

        def fire_inputs(m, p):
            pltpu.async_copy(cols_r.at[m], idx_v.at[p], sem_in)
            pltpu.async_copy(rows_r.at[m], row_v.at[p], sem_in)
            pltpu.async_copy(vals_r.at[m], val_v.at[p], sem_in)

        def wait_inputs(p):
            pltpu.make_async_copy(cols_r.at[0], idx_v.at[p], sem_in).wait()
            pltpu.make_async_copy(rows_r.at[0], row_v.at[p], sem_in).wait()
            pltpu.make_async_copy(vals_r.at[0], val_v.at[p], sem_in).wait()

        fire_inputs(m0, 0)

        def chunk(j, carry):
            p = lax.rem(j, 2)
            wait_inputs(p)
            mnext = jnp.where(j + 1 < nm, m0 + j + 1, m0 + j)
            fire_inputs(mnext, 1 - p)
            gd = [pltpu.async_copy(table_r.at[idx_v.at[p, b]],
                                   msg_v.at[pl.ds(b * _C, _C), :], sem_g)
                  for b in range(_NB)]
            sd = []
            for b in range(_NB):
                gd[b].wait()

                def mul(q, c2, b=b):
                    vv = val_v[p, b, pl.ds(q * 16, 16)]
                    for l in range(16):
                        i = b * _C + q * 16 + l
                        msg_v[i, :] = msg_v[i, :] * vv[l]
                    return c2
                lax.fori_loop(0, _C // 16, mul, 0)
                sd.append(pltpu.async_copy(msg_v.at[pl.ds(b * _C, _C), :],
                                           acc.at[row_v.at[p, b]],
                                           sem_s, add=True))
            for b in range(_NB):
                sd[b].wait()
            return carry

        lax.fori_loop(0, nm, chunk, 0)
        wait_inputs(lax.rem(jnp.int32(nm), 2))
        plsc.subcore_barrier()
        pltpu.sync_copy(acc.at[pl.ds(sid * slab, slab), :],
                        out_r.at[cid, sid])

    out = spmm(table, cols, rows, vals, zeros.reshape(_NS, slab, _EMB))
    return out.reshape(_NC, n, _EMB)


def _mid_tc(hp8, b1t, w2bd, n):
    n8 = n // 8
    r = w2bd.shape[0]

    def body(hp_ref, b1_ref, w2_ref, g_ref):
        h = jnp.maximum(hp_ref[0] + hp_ref[1] + b1_ref[:, :], 0.0)
        g_ref[0] = jnp.dot(h, w2_ref[0], preferred_element_type=jnp.float32)

    return pl.pallas_call(
        body,
        grid=(r,),
        in_specs=[
            pl.BlockSpec((2, n8, 128), lambda j: (0, 0, 0)),
            pl.BlockSpec((1, 128), lambda j: (0, 0)),
            pl.BlockSpec((1, 128, 128), lambda j: (j, 0, 0)),
        ],
        out_specs=pl.BlockSpec((1, n8, 128), lambda j: (j, 0, 0)),
        out_shape=jax.ShapeDtypeStruct((r, n8, 128), jnp.float32),
    )(hp8, b1t.reshape(1, 128), w2bd)


def _final_tc(op8, b2t, n):
    n8 = n // 8

    def body(p_ref, b2_ref, o_ref):
        o_ref[:, :] = p_ref[0] + p_ref[1] + b2_ref[:, :]

    return pl.pallas_call(
        body,
        grid=(1,),
        in_specs=[
            pl.BlockSpec((2, n8, 128), lambda i: (0, 0, 0)),
            pl.BlockSpec((1, 128), lambda i: (0, 0)),
        ],
        out_specs=pl.BlockSpec((n8, 128), lambda i: (0, 0)),
        out_shape=jax.ShapeDtypeStruct((n8, 128), jnp.float32),
    )(op8, b2t.reshape(1, 128))


def kernel(weights1, weights2, bias1, bias2, vals,
           hor_rows, hor_cols, ver_rows, ver_cols):
    r, n, emb = weights1.shape
    k = hor_rows.shape[0]
    wflat = weights1.reshape(r * n, emb)

    cpb = _NW * _MC
    kp = ((k + cpb - 1) // cpb) * cpb
    pad = kp - k
    pad_idx = jnp.arange(pad, dtype=jnp.int32) % n
    cols = jnp.concatenate([hor_cols.astype(jnp.int32),
                            pad_idx]).reshape(-1, _NB, _C)
    rows = jnp.concatenate([hor_rows.astype(jnp.int32),
                            pad_idx]).reshape(-1, _NB, _C)
    v = jnp.concatenate([vals.astype(jnp.float32),
                         jnp.zeros((pad,), jnp.float32)]).reshape(-1, _NB, _C)
    zeros = jnp.zeros((n, emb), jnp.float32)
    b1t = jnp.tile(bias1, 8)
    b2t = jnp.tile(bias2, 8)
    w2bd = jax.vmap(lambda w: jnp.kron(jnp.eye(8, dtype=w.dtype), w))(weights2)

    hp = _sc_spmm(wflat, cols, rows, v, zeros, n)
    g8 = _mid_tc(hp.reshape(_NC, n // 8, 128), b1t, w2bd, n)
    op = _sc_spmm(g8.reshape(r * n, emb), cols, rows, v, zeros, n)
    out8 = _final_tc(op.reshape(_NC, n // 8, 128), b2t, n)
    return out8.reshape(n, emb)
